# Initial kernel scaffold; baseline (speedup 1.0000x reference)
import functools

import jax
import jax.numpy as jnp
from jax import lax
from jax.experimental import pallas as pl
from jax.experimental.pallas import tpu as pltpu
from jax.experimental.pallas import tpu_sc as plsc

BATCH = 128
ZCH = 2000
NSUB = 16
F32 = jnp.float32


def _leaky(x):
    return jnp.where(x >= 0, x, 0.2 * x)


def _sc_edges_headsplit(src, dst, xl_flat, as_flat, ad_flat, zrows, zvec, N):
    E = src.shape[0]
    nb_total = E // BATCH
    per = nb_total // NSUB
    rem = nb_total - per * NSUB
    n_chunks = N // ZCH
    zfull = n_chunks // NSUB
    zrem = n_chunks - zfull * NSUB
    mesh = plsc.VectorSubcoreMesh(core_axis_name="c", subcore_axis_name="s")

    @functools.partial(
        pl.kernel,
        out_type=(jax.ShapeDtypeStruct((2 * N, 16), F32),
                  jax.ShapeDtypeStruct((2 * N,), F32)),
        mesh=mesh,
        scratch_types=[
            pltpu.VMEM((BATCH,), jnp.int32),
            pltpu.VMEM((BATCH,), jnp.int32),
            pltpu.VMEM((BATCH,), jnp.int32),
            pltpu.VMEM((BATCH,), jnp.int32),
            pltpu.VMEM((BATCH,), F32),
            pltpu.VMEM((BATCH,), F32),
            pltpu.VMEM((BATCH,), F32),
            pltpu.VMEM((BATCH, 16), F32),
            pltpu.VMEM_SHARED((N, 16), F32),
            pltpu.VMEM_SHARED((N,), F32),
        ],
    )
    def k(src_h, dst_h, xl_h, as_h, ad_h, zr_h, zv_h, acc_out, den_out,
          si, di, sio, dio, asv, adv, sv, rows, acc_sh, den_sh):
        c = lax.axis_index("c")
        sid = lax.axis_index("s")
        cN = c * N

        nz = zfull + jnp.where(sid < zrem, 1, 0)

        def zbody(t, _):
            off = (sid + NSUB * t) * ZCH
            pltpu.sync_copy(zr_h, acc_sh.at[pl.ds(off, ZCH)])
            pltpu.sync_copy(zv_h, den_sh.at[pl.ds(off, ZCH)])
            return 0

        lax.fori_loop(0, nz, zbody, 0)
        plsc.subcore_barrier()

        nb = per + jnp.where(sid < rem, 1, 0)
        start = sid * per + jnp.minimum(sid, rem)

        def ebody(j, _):
            eoff = (start + j) * BATCH
            pltpu.sync_copy(src_h.at[pl.ds(eoff, BATCH)], si)
            pltpu.sync_copy(dst_h.at[pl.ds(eoff, BATCH)], di)
            for q in range(BATCH // 16):
                sl = pl.ds(q * 16, 16)
                sio[sl] = si[sl] + cN
                dio[sl] = di[sl] + cN
            pltpu.sync_copy(as_h.at[sio], asv)
            pltpu.sync_copy(ad_h.at[dio], adv)
            pltpu.sync_copy(xl_h.at[sio], rows)
            for q in range(BATCH // 16):
                sl = pl.ds(q * 16, 16)
                sv[sl] = jnp.exp(_leaky(asv[sl] + adv[sl]))

            def mbody(e, _):
                rows[e, :] = rows[e, :] * sv[e]
                return 0

            lax.fori_loop(0, BATCH, mbody, 0)
            pltpu.sync_copy(sv, den_sh.at[di], add=True)
            pltpu.sync_copy(rows, acc_sh.at[di], add=True)
            return 0

        lax.fori_loop(0, nb, ebody, 0)
        plsc.subcore_barrier()

        def obody(t, _):
            off = (sid + NSUB * t) * ZCH
            pltpu.sync_copy(acc_sh.at[pl.ds(off, ZCH)],
                            acc_out.at[pl.ds(cN + off, ZCH)])
            pltpu.sync_copy(den_sh.at[pl.ds(off, ZCH)],
                            den_out.at[pl.ds(cN + off, ZCH)])
            return 0

        lax.fori_loop(0, nz, obody, 0)

    return k(src, dst, xl_flat, as_flat, ad_flat, zrows, zvec)


def _sc_edges_dstsplit(src, dst, xl, as_t, ad_t, zrows, zvec, N):
    E = src.shape[0]
    nb_total = E // BATCH
    per = nb_total // NSUB
    rem = nb_total - per * NSUB
    half = N // 2
    n_chunks = half // ZCH
    zfull = n_chunks // NSUB
    zrem = n_chunks - zfull * NSUB
    mesh = plsc.VectorSubcoreMesh(core_axis_name="c", subcore_axis_name="s")

    @functools.partial(
        pl.kernel,
        out_type=(jax.ShapeDtypeStruct((N, 16), F32),
                  jax.ShapeDtypeStruct((N,), F32)),
        mesh=mesh,
        scratch_types=[
            pltpu.VMEM((BATCH,), jnp.int32),
            pltpu.VMEM((BATCH,), jnp.int32),
            pltpu.VMEM((BATCH,), jnp.int32),
            pltpu.VMEM((BATCH,), F32),
            pltpu.VMEM((BATCH,), F32),
            pltpu.VMEM((BATCH,), F32),
            pltpu.VMEM((BATCH, 16), F32),
            pltpu.VMEM_SHARED((half + 8, 16), F32),
            pltpu.VMEM_SHARED((half + 16,), F32),
        ],
    )
    def k(src_h, dst_h, xl_h, as_h, ad_h, zr_h, zv_h, acc_out, den_out,
          si, di, dil, asv, adv, sv, rows, acc_sh, den_sh):
        c = lax.axis_index("c")
        sid = lax.axis_index("s")
        base = c * half

        nz = zfull + jnp.where(sid < zrem, 1, 0)

        def zbody(t, _):
            off = (sid + NSUB * t) * ZCH
            pltpu.sync_copy(zr_h, acc_sh.at[pl.ds(off, ZCH)])
            pltpu.sync_copy(zv_h, den_sh.at[pl.ds(off, ZCH)])
            return 0

        lax.fori_loop(0, nz, zbody, 0)

        @pl.when(sid == NSUB - 1)
        def _zero_pad():
            pltpu.sync_copy(zr_h.at[pl.ds(0, 8)], acc_sh.at[pl.ds(half, 8)])
            pltpu.sync_copy(zv_h.at[pl.ds(0, 16)], den_sh.at[pl.ds(half, 16)])

        plsc.subcore_barrier()

        nb = per + jnp.where(sid < rem, 1, 0)
        start = sid * per + jnp.minimum(sid, rem)

        def ebody(j, _):
            eoff = (start + j) * BATCH
            pltpu.sync_copy(src_h.at[pl.ds(eoff, BATCH)], si)
            pltpu.sync_copy(dst_h.at[pl.ds(eoff, BATCH)], di)
            pltpu.sync_copy(as_h.at[si], asv)
            pltpu.sync_copy(ad_h.at[di], adv)
            pltpu.sync_copy(xl_h.at[si], rows)
            for q in range(BATCH // 16):
                sl = pl.ds(q * 16, 16)
                sv[sl] = jnp.exp(_leaky(asv[sl] + adv[sl]))
                d = di[sl] - base
                ok = (d >= 0) & (d < half)
                dil[sl] = jnp.where(ok, d, half)

            def mbody(e, _):
                rows[e, :] = rows[e, :] * sv[e]
                return 0

            lax.fori_loop(0, BATCH, mbody, 0)
            pltpu.sync_copy(sv, den_sh.at[dil], add=True)
            pltpu.sync_copy(rows, acc_sh.at[dil], add=True)
            return 0

        lax.fori_loop(0, nb, ebody, 0)
        plsc.subcore_barrier()

        def obody(t, _):
            off = (sid + NSUB * t) * ZCH
            pltpu.sync_copy(acc_sh.at[pl.ds(off, ZCH)],
                            acc_out.at[pl.ds(base + off, ZCH)])
            pltpu.sync_copy(den_sh.at[pl.ds(off, ZCH)],
                            den_out.at[pl.ds(base + off, ZCH)])
            return 0

        lax.fori_loop(0, nz, obody, 0)

    return k(src, dst, xl, as_t, ad_t, zrows, zvec)


_TCB = 2000


def _tc_prep1(x, W1, a_src1, a_dst1):
    N = x.shape[0]

    def body(x_ref, w_ref, asr, adr, xl_ref, as_ref, ad_ref):
        xl = jnp.dot(x_ref[...], w_ref[...], preferred_element_type=F32)
        for h in range(2):
            xh = xl[:, h * 16:(h + 1) * 16]
            xl_ref[h, :, :] = xh
            as_ref[h, :] = (xh * asr[h, :][None, :]).sum(axis=1)
            ad_ref[h, :] = (xh * adr[h, :][None, :]).sum(axis=1)

    return pl.pallas_call(
        body,
        grid=(N // _TCB,),
        in_specs=[pl.BlockSpec((_TCB, 3), lambda i: (i, 0)),
                  pl.BlockSpec((3, 32), lambda i: (0, 0)),
                  pl.BlockSpec((2, 16), lambda i: (0, 0)),
                  pl.BlockSpec((2, 16), lambda i: (0, 0))],
        out_specs=[pl.BlockSpec((2, _TCB, 16), lambda i: (0, i, 0)),
                   pl.BlockSpec((2, _TCB), lambda i: (0, i)),
                   pl.BlockSpec((2, _TCB), lambda i: (0, i))],
        out_shape=[jax.ShapeDtypeStruct((2, N, 16), F32),
                   jax.ShapeDtypeStruct((2, N), F32),
                   jax.ShapeDtypeStruct((2, N), F32)],
    )(x, W1, a_src1, a_dst1)


def _tc_mid(acc, den, xl_prev, a_s, a_d, b, W, a_src_n, a_dst_n, Hn):
    N = acc.shape[1]

    def body(acc_ref, den_ref, xlp_ref, as_ref, ad_ref, b_ref, w_ref,
             asn_ref, adn_ref, xln_ref, asno_ref, adno_ref):
        outs = []
        for h in range(2):
            ss = jnp.exp(_leaky(as_ref[h, :] + ad_ref[h, :]))
            at = acc_ref[h, :, :] + xlp_ref[h, :, :] * ss[:, None]
            dt = den_ref[h, :] + ss
            outs.append(at / (dt[:, None] + 1e-16))
        hcat = jnp.concatenate(outs, axis=1) + b_ref[0, :][None, :]
        hcat = jnp.where(hcat > 0, hcat, jnp.expm1(hcat))
        xln = jnp.dot(hcat, w_ref[...], preferred_element_type=F32)
        for h in range(Hn):
            xh = xln[:, h * 16:(h + 1) * 16]
            xln_ref[h, :, :] = xh
            asno_ref[h, :] = (xh * asn_ref[h, :][None, :]).sum(axis=1)
            adno_ref[h, :] = (xh * adn_ref[h, :][None, :]).sum(axis=1)

    Cn = W.shape[1]
    return pl.pallas_call(
        body,
        grid=(N // _TCB,),
        in_specs=[pl.BlockSpec((2, _TCB, 16), lambda i: (0, i, 0)),
                  pl.BlockSpec((2, _TCB), lambda i: (0, i)),
                  pl.BlockSpec((2, _TCB, 16), lambda i: (0, i, 0)),
                  pl.BlockSpec((2, _TCB), lambda i: (0, i)),
                  pl.BlockSpec((2, _TCB), lambda i: (0, i)),
                  pl.BlockSpec((1, 32), lambda i: (0, 0)),
                  pl.BlockSpec((32, Cn), lambda i: (0, 0)),
                  pl.BlockSpec((Hn, 16), lambda i: (0, 0)),
                  pl.BlockSpec((Hn, 16), lambda i: (0, 0))],
        out_specs=[pl.BlockSpec((Hn, _TCB, 16), lambda i: (0, i, 0)),
                   pl.BlockSpec((Hn, _TCB), lambda i: (0, i)),
                   pl.BlockSpec((Hn, _TCB), lambda i: (0, i))],
        out_shape=[jax.ShapeDtypeStruct((Hn, N, 16), F32),
                   jax.ShapeDtypeStruct((Hn, N), F32),
                   jax.ShapeDtypeStruct((Hn, N), F32)],
    )(acc, den, xl_prev, a_s, a_d, b, W, a_src_n, a_dst_n)


def _tc_final(acc, den, xl3, a_s3, a_d3, b3, Wo, bo):
    N = acc.shape[0]

    def body(acc_ref, den_ref, xl_ref, as_ref, ad_ref, b_ref, wo_ref, bo_ref,
             out_ref, emb_ref):
        ss = jnp.exp(_leaky(as_ref[0, :] + ad_ref[0, :]))
        at = acc_ref[...] + xl_ref[...] * ss[:, None]
        dt = den_ref[0, :] + ss
        h = at / (dt[:, None] + 1e-16) + b_ref[0, :][None, :]
        h = jnp.where(h > 0, h, jnp.expm1(h))
        emb_ref[...] = h
        z = jnp.dot(h, wo_ref[...], preferred_element_type=F32) + bo_ref[0, :]
        out_ref[...] = 1.0 / (1.0 + jnp.exp(-z))

    return pl.pallas_call(
        body,
        grid=(N // _TCB,),
        in_specs=[pl.BlockSpec((_TCB, 16), lambda i: (i, 0)),
                  pl.BlockSpec((1, _TCB), lambda i: (0, i)),
                  pl.BlockSpec((_TCB, 16), lambda i: (i, 0)),
                  pl.BlockSpec((1, _TCB), lambda i: (0, i)),
                  pl.BlockSpec((1, _TCB), lambda i: (0, i)),
                  pl.BlockSpec((1, 16), lambda i: (0, 0)),
                  pl.BlockSpec((16, 1), lambda i: (0, 0)),
                  pl.BlockSpec((1, 1), lambda i: (0, 0))],
        out_specs=[pl.BlockSpec((_TCB, 1), lambda i: (i, 0)),
                   pl.BlockSpec((_TCB, 16), lambda i: (i, 0))],
        out_shape=[jax.ShapeDtypeStruct((N, 1), F32),
                   jax.ShapeDtypeStruct((N, 16), F32)],
    )(acc, den, xl3, a_s3, a_d3, b3, Wo, bo)


def kernel(x, edge_index, W1, a_src1, a_dst1, b1, W2, a_src2, a_dst2, b2,
           W3, a_src3, a_dst3, b3, Wo, bo):
    N = x.shape[0]
    ei = edge_index.astype(jnp.int32)
    src, dst = ei[0], ei[1]
    zrows = jnp.zeros((ZCH, 16), F32)
    zvec = jnp.zeros((ZCH,), F32)

    xl1, as1, ad1 = _tc_prep1(x, W1, a_src1, a_dst1)
    acc1, den1 = _sc_edges_headsplit(
        src, dst, xl1.reshape(2 * N, 16), as1.reshape(2 * N),
        ad1.reshape(2 * N), zrows, zvec, N)
    acc1 = acc1.reshape(2, N, 16)
    den1 = den1.reshape(2, N)

    xl2, as2, ad2 = _tc_mid(acc1, den1, xl1, as1, ad1, b1.reshape(1, 32),
                            W2, a_src2, a_dst2, 2)
    acc2, den2 = _sc_edges_headsplit(
        src, dst, xl2.reshape(2 * N, 16), as2.reshape(2 * N),
        ad2.reshape(2 * N), zrows, zvec, N)
    acc2 = acc2.reshape(2, N, 16)
    den2 = den2.reshape(2, N)

    xl3, as3, ad3 = _tc_mid(acc2, den2, xl2, as2, ad2, b2.reshape(1, 32),
                            W3, a_src3, a_dst3, 1)
    xl3f = xl3.reshape(N, 16)
    acc3, den3 = _sc_edges_dstsplit(
        src, dst, xl3f, as3.reshape(N), ad3.reshape(N), zrows, zvec, N)

    out, emb = _tc_final(acc3, den3.reshape(1, N), xl3f,
                         as3.reshape(1, N), ad3.reshape(1, N),
                         b3.reshape(1, 16), Wo, bo.reshape(1, 1))
    return (out, emb)


# speedup vs baseline: 46.8585x; 46.8585x over previous
import functools

import jax
import jax.numpy as jnp
from jax import lax
from jax.experimental import pallas as pl
from jax.experimental.pallas import tpu as pltpu
from jax.experimental.pallas import tpu_sc as plsc

BATCH = 128
ZCH = 2000
NSUB = 16
F32 = jnp.float32


def _leaky(x):
    return jnp.where(x >= 0, x, 0.2 * x)


def _sc_edges_headsplit(src, dst, xl_flat, as_flat, ad_flat, zrows, N):
    E = src.shape[0]
    nb_total = E // BATCH
    per = nb_total // NSUB
    rem = nb_total - per * NSUB
    n_chunks = N // ZCH
    zfull = n_chunks // NSUB
    zrem = n_chunks - zfull * NSUB
    mesh = plsc.VectorSubcoreMesh(core_axis_name="c", subcore_axis_name="s")

    @functools.partial(
        pl.kernel,
        out_type=(jax.ShapeDtypeStruct((2 * N, 16), F32),
                  jax.ShapeDtypeStruct((2 * N // ZCH, ZCH), F32)),
        mesh=mesh,
        compiler_params=pltpu.CompilerParams(use_tc_tiling_on_sc=False),
        scratch_types=[
            pltpu.VMEM((BATCH,), jnp.int32),
            pltpu.VMEM((BATCH,), jnp.int32),
            pltpu.VMEM((BATCH,), jnp.int32),
            pltpu.VMEM((BATCH,), jnp.int32),
            pltpu.VMEM((BATCH,), F32),
            pltpu.VMEM((BATCH,), F32),
            pltpu.VMEM((BATCH,), F32),
            pltpu.VMEM((BATCH, 16), F32),
            pltpu.VMEM((ZCH,), F32),
            pltpu.VMEM_SHARED((N, 16), F32),
            pltpu.VMEM_SHARED((N,), F32),
        ],
    )
    def k(src_h, dst_h, xl_h, as_h, ad_h, zr_h, acc_out, den_out,
          si, di, sio, dio, asv, adv, sv, rows, zb, acc_sh, den_sh):
        c = lax.axis_index("c")
        sid = lax.axis_index("s")
        cN = c * N
        for q in range(ZCH // 16):
            zb[pl.ds(q * 16, 16)] = jnp.zeros((16,), F32)

        nz = zfull + jnp.where(sid < zrem, 1, 0)

        def zbody(t, _):
            off = (sid + NSUB * t) * ZCH
            pltpu.sync_copy(zr_h, acc_sh.at[pl.ds(off, ZCH)])
            pltpu.sync_copy(zb, den_sh.at[pl.ds(off, ZCH)])
            return 0

        lax.fori_loop(0, nz, zbody, 0)
        plsc.subcore_barrier()

        nb = per + jnp.where(sid < rem, 1, 0)
        start = sid * per + jnp.minimum(sid, rem)

        def ebody(j, _):
            eoff = (start + j) * BATCH
            pltpu.sync_copy(src_h.at[pl.ds(eoff, BATCH)], si)
            pltpu.sync_copy(dst_h.at[pl.ds(eoff, BATCH)], di)
            for q in range(BATCH // 16):
                sl = pl.ds(q * 16, 16)
                sio[sl] = si[sl] + cN
                dio[sl] = di[sl] + cN
            pltpu.sync_copy(as_h.at[sio], asv)
            pltpu.sync_copy(ad_h.at[dio], adv)
            pltpu.sync_copy(xl_h.at[sio], rows)
            for q in range(BATCH // 16):
                sl = pl.ds(q * 16, 16)
                sv[sl] = jnp.exp(_leaky(asv[sl] + adv[sl]))

            def mbody(g, _):
                svec = sv[pl.ds(g * 16, 16)]
                base2 = g * 16
                for l in range(16):
                    rows[base2 + l, :] = rows[base2 + l, :] * svec[l]
                return 0

            lax.fori_loop(0, BATCH // 16, mbody, 0)
            pltpu.sync_copy(sv, den_sh.at[di], add=True)
            pltpu.sync_copy(rows, acc_sh.at[di], add=True)
            return 0

        lax.fori_loop(0, nb, ebody, 0)
        plsc.subcore_barrier()

        def obody(t, _):
            off = (sid + NSUB * t) * ZCH
            pltpu.sync_copy(acc_sh.at[pl.ds(off, ZCH)],
                            acc_out.at[pl.ds(cN + off, ZCH)])
            pltpu.sync_copy(den_sh.at[pl.ds(off, ZCH)],
                            den_out.at[(cN + off) // ZCH])
            return 0

        lax.fori_loop(0, nz, obody, 0)

    return k(src, dst, xl_flat, as_flat, ad_flat, zrows)


def _sc_edges_dstsplit(src, dst, xl, as_t, ad_t, zrows, N):
    E = src.shape[0]
    nb_total = E // BATCH
    per = nb_total // NSUB
    rem = nb_total - per * NSUB
    half = N // 2
    n_chunks = half // ZCH
    zfull = n_chunks // NSUB
    zrem = n_chunks - zfull * NSUB
    mesh = plsc.VectorSubcoreMesh(core_axis_name="c", subcore_axis_name="s")

    @functools.partial(
        pl.kernel,
        out_type=(jax.ShapeDtypeStruct((N, 16), F32),
                  jax.ShapeDtypeStruct((N // ZCH, ZCH), F32)),
        mesh=mesh,
        compiler_params=pltpu.CompilerParams(use_tc_tiling_on_sc=False),
        scratch_types=[
            pltpu.VMEM((BATCH,), jnp.int32),
            pltpu.VMEM((BATCH,), jnp.int32),
            pltpu.VMEM((BATCH,), jnp.int32),
            pltpu.VMEM((BATCH,), F32),
            pltpu.VMEM((BATCH,), F32),
            pltpu.VMEM((BATCH,), F32),
            pltpu.VMEM((BATCH, 16), F32),
            pltpu.VMEM((ZCH,), F32),
            pltpu.VMEM_SHARED((half + 8, 16), F32),
            pltpu.VMEM_SHARED((half + 16,), F32),
        ],
    )
    def k(src_h, dst_h, xl_h, as_h, ad_h, zr_h, acc_out, den_out,
          si, di, dil, asv, adv, sv, rows, zb, acc_sh, den_sh):
        c = lax.axis_index("c")
        sid = lax.axis_index("s")
        base = c * half
        for q in range(ZCH // 16):
            zb[pl.ds(q * 16, 16)] = jnp.zeros((16,), F32)

        nz = zfull + jnp.where(sid < zrem, 1, 0)

        def zbody(t, _):
            off = (sid + NSUB * t) * ZCH
            pltpu.sync_copy(zr_h, acc_sh.at[pl.ds(off, ZCH)])
            pltpu.sync_copy(zb, den_sh.at[pl.ds(off, ZCH)])
            return 0

        lax.fori_loop(0, nz, zbody, 0)

        @pl.when(sid == NSUB - 1)
        def _zero_pad():
            pltpu.sync_copy(zr_h.at[pl.ds(0, 8)], acc_sh.at[pl.ds(half, 8)])
            pltpu.sync_copy(zb.at[pl.ds(0, 16)], den_sh.at[pl.ds(half, 16)])

        plsc.subcore_barrier()

        nb = per + jnp.where(sid < rem, 1, 0)
        start = sid * per + jnp.minimum(sid, rem)

        def ebody(j, _):
            eoff = (start + j) * BATCH
            pltpu.sync_copy(src_h.at[pl.ds(eoff, BATCH)], si)
            pltpu.sync_copy(dst_h.at[pl.ds(eoff, BATCH)], di)
            pltpu.sync_copy(as_h.at[si], asv)
            pltpu.sync_copy(ad_h.at[di], adv)
            pltpu.sync_copy(xl_h.at[si], rows)
            for q in range(BATCH // 16):
                sl = pl.ds(q * 16, 16)
                sv[sl] = jnp.exp(_leaky(asv[sl] + adv[sl]))
                d = di[sl] - base
                ok = (d >= 0) & (d < half)
                dil[sl] = jnp.where(ok, d, half)

            def mbody(g, _):
                svec = sv[pl.ds(g * 16, 16)]
                base2 = g * 16
                for l in range(16):
                    rows[base2 + l, :] = rows[base2 + l, :] * svec[l]
                return 0

            lax.fori_loop(0, BATCH // 16, mbody, 0)
            pltpu.sync_copy(sv, den_sh.at[dil], add=True)
            pltpu.sync_copy(rows, acc_sh.at[dil], add=True)
            return 0

        lax.fori_loop(0, nb, ebody, 0)
        plsc.subcore_barrier()

        def obody(t, _):
            off = (sid + NSUB * t) * ZCH
            pltpu.sync_copy(acc_sh.at[pl.ds(off, ZCH)],
                            acc_out.at[pl.ds(base + off, ZCH)])
            pltpu.sync_copy(den_sh.at[pl.ds(off, ZCH)],
                            den_out.at[(base + off) // ZCH])
            return 0

        lax.fori_loop(0, nz, obody, 0)

    return k(src, dst, xl, as_t, ad_t, zrows)


_TCB = 2000


def _tc_prep1(x, W1, a_src1, a_dst1):
    N = x.shape[0]

    def body(x_ref, w_ref, asr, adr, xl_ref, as_ref, ad_ref):
        xl = jnp.dot(x_ref[...], w_ref[...], preferred_element_type=F32)
        for h in range(2):
            xh = xl[:, h * 16:(h + 1) * 16]
            xl_ref[h, :, :] = xh
            as_ref[h, :, :] = (xh * asr[h, :][None, :]).sum(1, keepdims=True)
            ad_ref[h, :, :] = (xh * adr[h, :][None, :]).sum(1, keepdims=True)

    return pl.pallas_call(
        body,
        grid=(N // _TCB,),
        in_specs=[pl.BlockSpec((_TCB, 3), lambda i: (i, 0)),
                  pl.BlockSpec((3, 32), lambda i: (0, 0)),
                  pl.BlockSpec((2, 16), lambda i: (0, 0)),
                  pl.BlockSpec((2, 16), lambda i: (0, 0))],
        out_specs=[pl.BlockSpec((2, _TCB, 16), lambda i: (0, i, 0)),
                   pl.BlockSpec((2, _TCB, 1), lambda i: (0, i, 0)),
                   pl.BlockSpec((2, _TCB, 1), lambda i: (0, i, 0))],
        out_shape=[jax.ShapeDtypeStruct((2, N, 16), F32),
                   jax.ShapeDtypeStruct((2, N, 1), F32),
                   jax.ShapeDtypeStruct((2, N, 1), F32)],
    )(x, W1, a_src1, a_dst1)


def _tc_mid(acc, den, xl_prev, a_s, a_d, b, W, a_src_n, a_dst_n, Hn):
    N = acc.shape[1]

    def body(acc_ref, den_ref, xlp_ref, as_ref, ad_ref, b_ref, w_ref,
             asn_ref, adn_ref, xln_ref, asno_ref, adno_ref):
        outs = []
        for h in range(2):
            ss = jnp.exp(_leaky(as_ref[h, :, :] + ad_ref[h, :, :]))
            at = acc_ref[h, :, :] + xlp_ref[h, :, :] * ss
            dt = den_ref[h, :, :] + ss
            outs.append(at / (dt + 1e-16))
        hcat = jnp.concatenate(outs, axis=1) + b_ref[0, :][None, :]
        hcat = jnp.where(hcat > 0, hcat, jnp.exp(hcat) - 1.0)
        xln = jnp.dot(hcat, w_ref[...], preferred_element_type=F32)
        for h in range(Hn):
            xh = xln[:, h * 16:(h + 1) * 16]
            xln_ref[h, :, :] = xh
            asno_ref[h, :, :] = (xh * asn_ref[h, :][None, :]).sum(1, keepdims=True)
            adno_ref[h, :, :] = (xh * adn_ref[h, :][None, :]).sum(1, keepdims=True)

    Cn = W.shape[1]
    return pl.pallas_call(
        body,
        grid=(N // _TCB,),
        in_specs=[pl.BlockSpec((2, _TCB, 16), lambda i: (0, i, 0)),
                  pl.BlockSpec((2, _TCB, 1), lambda i: (0, i, 0)),
                  pl.BlockSpec((2, _TCB, 16), lambda i: (0, i, 0)),
                  pl.BlockSpec((2, _TCB, 1), lambda i: (0, i, 0)),
                  pl.BlockSpec((2, _TCB, 1), lambda i: (0, i, 0)),
                  pl.BlockSpec((1, 32), lambda i: (0, 0)),
                  pl.BlockSpec((32, Cn), lambda i: (0, 0)),
                  pl.BlockSpec((Hn, 16), lambda i: (0, 0)),
                  pl.BlockSpec((Hn, 16), lambda i: (0, 0))],
        out_specs=[pl.BlockSpec((Hn, _TCB, 16), lambda i: (0, i, 0)),
                   pl.BlockSpec((Hn, _TCB, 1), lambda i: (0, i, 0)),
                   pl.BlockSpec((Hn, _TCB, 1), lambda i: (0, i, 0))],
        out_shape=[jax.ShapeDtypeStruct((Hn, N, 16), F32),
                   jax.ShapeDtypeStruct((Hn, N, 1), F32),
                   jax.ShapeDtypeStruct((Hn, N, 1), F32)],
    )(acc, den, xl_prev, a_s, a_d, b, W, a_src_n, a_dst_n)


def _tc_final(acc, den, xl3, a_s3, a_d3, b3, Wo, bo):
    N = acc.shape[0]

    def body(acc_ref, den_ref, xl_ref, as_ref, ad_ref, b_ref, wo_ref, bo_ref,
             out_ref, emb_ref):
        ss = jnp.exp(_leaky(as_ref[...] + ad_ref[...]))
        at = acc_ref[...] + xl_ref[...] * ss
        dt = den_ref[...] + ss
        h = at / (dt + 1e-16) + b_ref[0, :][None, :]
        h = jnp.where(h > 0, h, jnp.exp(h) - 1.0)
        emb_ref[...] = h
        z = jnp.dot(h, wo_ref[...], preferred_element_type=F32) + bo_ref[0, :]
        out_ref[...] = 1.0 / (1.0 + jnp.exp(-z))

    return pl.pallas_call(
        body,
        grid=(N // _TCB,),
        in_specs=[pl.BlockSpec((_TCB, 16), lambda i: (i, 0)),
                  pl.BlockSpec((_TCB, 1), lambda i: (i, 0)),
                  pl.BlockSpec((_TCB, 16), lambda i: (i, 0)),
                  pl.BlockSpec((_TCB, 1), lambda i: (i, 0)),
                  pl.BlockSpec((_TCB, 1), lambda i: (i, 0)),
                  pl.BlockSpec((1, 16), lambda i: (0, 0)),
                  pl.BlockSpec((16, 1), lambda i: (0, 0)),
                  pl.BlockSpec((1, 1), lambda i: (0, 0))],
        out_specs=[pl.BlockSpec((_TCB, 1), lambda i: (i, 0)),
                   pl.BlockSpec((_TCB, 16), lambda i: (i, 0))],
        out_shape=[jax.ShapeDtypeStruct((N, 1), F32),
                   jax.ShapeDtypeStruct((N, 16), F32)],
    )(acc, den, xl3, a_s3, a_d3, b3, Wo, bo)


def kernel(x, edge_index, W1, a_src1, a_dst1, b1, W2, a_src2, a_dst2, b2,
           W3, a_src3, a_dst3, b3, Wo, bo):
    N = x.shape[0]
    ei = edge_index.astype(jnp.int32)
    src, dst = ei[0], ei[1]
    zrows = jnp.zeros((ZCH, 16), F32)

    xl1, as1, ad1 = _tc_prep1(x, W1, a_src1, a_dst1)
    acc1, den1 = _sc_edges_headsplit(
        src, dst, xl1.reshape(2 * N, 16), as1.reshape(2 * N),
        ad1.reshape(2 * N), zrows, N)
    acc1 = acc1.reshape(2, N, 16)
    den1 = den1.reshape(2, N, 1)

    xl2, as2, ad2 = _tc_mid(acc1, den1, xl1, as1, ad1, b1.reshape(1, 32),
                            W2, a_src2, a_dst2, 2)
    acc2, den2 = _sc_edges_headsplit(
        src, dst, xl2.reshape(2 * N, 16), as2.reshape(2 * N),
        ad2.reshape(2 * N), zrows, N)
    acc2 = acc2.reshape(2, N, 16)
    den2 = den2.reshape(2, N, 1)

    xl3, as3, ad3 = _tc_mid(acc2, den2, xl2, as2, ad2, b2.reshape(1, 32),
                            W3, a_src3, a_dst3, 1)
    xl3f = xl3.reshape(N, 16)
    acc3, den3 = _sc_edges_dstsplit(
        src, dst, xl3f, as3.reshape(N), ad3.reshape(N), zrows, N)

    out, emb = _tc_final(acc3, den3.reshape(N, 1), xl3f,
                         as3.reshape(N, 1), ad3.reshape(N, 1),
                         b3.reshape(1, 16), Wo, bo.reshape(1, 1))
    return (out, emb)
